# Initial kernel scaffold; baseline (speedup 1.0000x reference)
import jax
import jax.numpy as jnp
from jax.experimental import pallas as pl


def kernel(dag_x, dag_edge_index, dag_layers_mask, dag_edge_attr, dag_readouts, dag_leaves, batch, Wt, bt, W1, b1, W2, b2, Wfc1, bfc1, Wfc2, bfc2):
    raise NotImplementedError("write your pallas kernel here")


# speedup vs baseline: 3.7019x; 3.7019x over previous
import functools

import jax
import jax.numpy as jnp
from jax import lax
from jax.experimental import pallas as pl
from jax.experimental.pallas import tpu as pltpu
from jax.experimental.pallas import tpu_sc as plsc

N, E, D, L, G, T, R = 10000, 160000, 256, 4, 64, 128, 2048
NP = 10240
H = D // 2
BN = 512
EB = 80
ER = E // EB
NT = 16
TPT = ER // NT
OCH = 25
NOU = TPT // OCH
STR = NP // NT
RR = (L + 1) * R // EB
RPT = RR // NT
OS = (L + 1) * G

f32 = jnp.float32
i32 = jnp.int32


@functools.cache
def _mesh():
    return plsc.VectorSubcoreMesh(core_axis_name="c", subcore_axis_name="s")


def _sc_init_body(lv_h, im_out, mv, lv):
    c = lax.axis_index("c")
    s = lax.axis_index("s")

    @pl.when(jnp.logical_and(c == 0, s == 0))
    def _():
        def _z(k, carry):
            mv[pl.ds(k * 16, 16)] = jnp.zeros((16,), f32)
            return carry
        lax.fori_loop(0, NP // 16, _z, None)
        pltpu.sync_copy(lv_h, lv)

        def _sc(g, carry):
            r16 = lv[pl.ds(g * 16, 16)]
            plsc.store_scatter(mv, [r16], jnp.ones((16,), f32))
            return carry
        lax.fori_loop(0, R // 16, _sc, None)
        pltpu.sync_copy(mv, im_out)


@functools.cache
def _sc_init():
    return pl.kernel(
        _sc_init_body,
        mesh=_mesh(),
        compiler_params=pltpu.CompilerParams(needs_layout_passes=False, use_tc_tiling_on_sc=False),
        out_type=jax.ShapeDtypeStruct((NP,), f32),
        scratch_types=[
            pltpu.VMEM((NP,), f32),
            pltpu.VMEM((R,), i32),
        ],
    )


def _sc_layer_body(xL, xR, src2, dst2, msk2, att2, li_h,
                   exL, exR, cnt,
                   srcv, dstv, mskv, attv, onesv, rows, liv, exsh, cntsh, sem):
    c = lax.axis_index("c")
    s = lax.axis_index("s")
    pltpu.sync_copy(li_h, liv)
    lid = liv[pl.ds(0, 16)][0]

    def _zrow(k, carry):
        rows[k // 8, pl.ds((k % 8) * 16, 16)] = jnp.zeros((16,), f32)
        return carry
    lax.fori_loop(0, EB * 8, _zrow, None)
    for k in range(STR // EB):
        pltpu.sync_copy(rows, exsh.at[pl.ds(s * STR + k * EB, EB)])
    for g in range(EB // 16):
        onesv[pl.ds(g * 16, 16)] = jnp.zeros((16,), f32)
    for k in range(STR // EB):
        pltpu.sync_copy(onesv, cntsh.at[pl.ds(s * STR + k * EB, EB)])
    plsc.subcore_barrier()

    def _outer(o, carry):
        rowbase = s * TPT + o * OCH
        pltpu.sync_copy(src2.at[pl.ds(rowbase, OCH)], srcv)
        pltpu.sync_copy(dst2.at[pl.ds(rowbase, OCH)], dstv)
        pltpu.sync_copy(msk2.at[pl.ds(rowbase, OCH)], mskv)
        pltpu.sync_copy(att2.at[pl.ds(rowbase, OCH)], attv)

        def _inner(j, icarry):
            @pl.when(c == 0)
            def _():
                pltpu.async_copy(xL.at[srcv.at[j]], rows, sem).wait()

            @pl.when(c == 1)
            def _():
                pltpu.async_copy(xR.at[srcv.at[j]], rows, sem).wait()

            for g in range(EB // 16):
                gsl = pl.ds(g * 16, 16)
                m16 = mskv[j, gsl] == lid
                w16 = jnp.where(m16, attv[j, gsl], 0.0)
                for l in range(16):
                    w = w16[l]
                    e = g * 16 + l
                    for cb in range(8):
                        sl = pl.ds(cb * 16, 16)
                        rows[e, sl] = rows[e, sl] * w

            @pl.when(c == 0)
            def _():
                for g in range(EB // 16):
                    sl = pl.ds(g * 16, 16)
                    m16 = mskv[j, sl] == lid
                    onesv[sl] = jnp.where(m16, 1.0, 0.0)
                pltpu.sync_copy(onesv, cntsh.at[dstv.at[j]], add=True)

            pltpu.sync_copy(rows, exsh.at[dstv.at[j]], add=True)
            return icarry
        lax.fori_loop(0, OCH, _inner, None)
        return carry
    lax.fori_loop(0, NOU, _outer, None)
    plsc.subcore_barrier()

    @pl.when(c == 0)
    def _():
        pltpu.sync_copy(exsh.at[pl.ds(s * STR, STR)], exL.at[pl.ds(s * STR, STR)])
        pltpu.sync_copy(cntsh.at[pl.ds(s * STR, STR)], cnt.at[pl.ds(s * STR, STR)])

    @pl.when(c == 1)
    def _():
        pltpu.sync_copy(exsh.at[pl.ds(s * STR, STR)], exR.at[pl.ds(s * STR, STR)])


@functools.cache
def _sc_layer():
  return pl.kernel(
    _sc_layer_body,
    mesh=_mesh(),
    compiler_params=pltpu.CompilerParams(needs_layout_passes=False, use_tc_tiling_on_sc=False),
    out_type=[
        jax.ShapeDtypeStruct((NP, H), f32),
        jax.ShapeDtypeStruct((NP, H), f32),
        jax.ShapeDtypeStruct((NP,), f32),
    ],
    scratch_types=[
        pltpu.VMEM((OCH, EB), i32),
        pltpu.VMEM((OCH, EB), i32),
        pltpu.VMEM((OCH, EB), i32),
        pltpu.VMEM((OCH, EB), f32),
        pltpu.VMEM((EB,), f32),
        pltpu.VMEM((EB, H), f32),
        pltpu.VMEM((16,), i32),
        pltpu.VMEM_SHARED((NP, H), f32),
        pltpu.VMEM_SHARED((NP,), f32),
        pltpu.SemaphoreType.DMA,
    ],
  )


def _sc_read_body(xL, xR, r2, batchp, oL, oR, rv, segv, rows, bv, osh, sem):
    c = lax.axis_index("c")
    s = lax.axis_index("s")
    pltpu.sync_copy(batchp, bv)
    pltpu.sync_copy(r2.at[pl.ds(s * RPT, RPT)], rv)

    def _zrow(k, carry):
        rows[k // 8, pl.ds((k % 8) * 16, 16)] = jnp.zeros((16,), f32)
        return carry
    lax.fori_loop(0, (OS // NT) * 8, _zrow, None)
    pltpu.sync_copy(rows.at[pl.ds(0, OS // NT)],
                    osh.at[pl.ds(s * (OS // NT), OS // NT)])
    plsc.subcore_barrier()

    for j in range(RPT):
        for g in range(EB // 16):
            sl = pl.ds(g * 16, 16)
            r16 = rv[j, sl]
            seg16 = plsc.load_gather(bv, [r16])
            flat = (s * RPT + j) * EB + g * 16
            segv[j, sl] = seg16 + (flat // R) * G

        @pl.when(c == 0)
        def _():
            pltpu.async_copy(xL.at[rv.at[j]], rows, sem).wait()

        @pl.when(c == 1)
        def _():
            pltpu.async_copy(xR.at[rv.at[j]], rows, sem).wait()

        pltpu.sync_copy(rows, osh.at[segv.at[j]], add=True)
    plsc.subcore_barrier()

    @pl.when(c == 0)
    def _():
        pltpu.sync_copy(osh.at[pl.ds(s * (OS // NT), OS // NT)],
                        oL.at[pl.ds(s * (OS // NT), OS // NT)])

    @pl.when(c == 1)
    def _():
        pltpu.sync_copy(osh.at[pl.ds(s * (OS // NT), OS // NT)],
                        oR.at[pl.ds(s * (OS // NT), OS // NT)])


@functools.cache
def _sc_read():
  return pl.kernel(
    _sc_read_body,
    mesh=_mesh(),
    compiler_params=pltpu.CompilerParams(needs_layout_passes=False, use_tc_tiling_on_sc=False),
    out_type=[
        jax.ShapeDtypeStruct((OS, H), f32),
        jax.ShapeDtypeStruct((OS, H), f32),
    ],
    scratch_types=[
        pltpu.VMEM((RPT, EB), i32),
        pltpu.VMEM((RPT, EB), i32),
        pltpu.VMEM((EB, H), f32),
        pltpu.VMEM((NP,), i32),
        pltpu.VMEM_SHARED((OS, H), f32),
        pltpu.SemaphoreType.DMA,
    ],
  )


def _prep_body(x_ref, w_ref, b_ref, m_ref, fL_ref, fR_ref, xL_ref, xR_ref):
    f = jnp.maximum(
        jnp.dot(x_ref[...], w_ref[...], preferred_element_type=f32) + b_ref[...],
        0.0)
    m = m_ref[...]
    fL_ref[...] = f[:, :H]
    fR_ref[...] = f[:, H:]
    xL_ref[...] = f[:, :H] * m
    xR_ref[...] = f[:, H:] * m


_tc_prep = pl.pallas_call(
    _prep_body,
    grid=(NP // BN,),
    in_specs=[
        pl.BlockSpec((BN, D), lambda i: (i, 0)),
        pl.BlockSpec((D, D), lambda i: (0, 0)),
        pl.BlockSpec((1, D), lambda i: (0, 0)),
        pl.BlockSpec((BN, 1), lambda i: (i, 0)),
    ],
    out_specs=[pl.BlockSpec((BN, H), lambda i: (i, 0))] * 4,
    out_shape=[jax.ShapeDtypeStruct((NP, H), f32)] * 4,
)


def _layer_body(cnt_ref, fL_ref, fR_ref, eL_ref, eR_ref, xL_ref, xR_ref,
                w1_ref, b1_ref, w2_ref, b2_ref, oL_ref, oR_ref):
    t = (cnt_ref[...] > 0.0).astype(f32)
    sL = eL_ref[...] + fL_ref[...] * t
    sR = eR_ref[...] + fR_ref[...] * t
    sfull = jnp.concatenate([sL, sR], axis=1)
    h = jnp.maximum(
        jnp.dot(sfull, w1_ref[...], preferred_element_type=f32) + b1_ref[...],
        0.0)
    h = jnp.maximum(
        jnp.dot(h, w2_ref[...], preferred_element_type=f32) + b2_ref[...],
        0.0)
    oL_ref[...] = xL_ref[...] + h[:, :H] * t
    oR_ref[...] = xR_ref[...] + h[:, H:] * t


_tc_layer = pl.pallas_call(
    _layer_body,
    grid=(NP // BN,),
    in_specs=[
        pl.BlockSpec((BN, 1), lambda i: (i, 0)),
        pl.BlockSpec((BN, H), lambda i: (i, 0)),
        pl.BlockSpec((BN, H), lambda i: (i, 0)),
        pl.BlockSpec((BN, H), lambda i: (i, 0)),
        pl.BlockSpec((BN, H), lambda i: (i, 0)),
        pl.BlockSpec((BN, H), lambda i: (i, 0)),
        pl.BlockSpec((BN, H), lambda i: (i, 0)),
        pl.BlockSpec((D, D), lambda i: (0, 0)),
        pl.BlockSpec((1, D), lambda i: (0, 0)),
        pl.BlockSpec((D, D), lambda i: (0, 0)),
        pl.BlockSpec((1, D), lambda i: (0, 0)),
    ],
    out_specs=[pl.BlockSpec((BN, H), lambda i: (i, 0))] * 2,
    out_shape=[jax.ShapeDtypeStruct((NP, H), f32)] * 2,
)


def _final_body(x_ref, w1_ref, b1_ref, w2_ref, b2_ref, o_ref):
    h = jnp.maximum(
        jnp.dot(x_ref[...], w1_ref[...], preferred_element_type=f32) + b1_ref[...],
        0.0)
    o_ref[...] = jnp.dot(h, w2_ref[...], preferred_element_type=f32) + b2_ref[...]


_tc_final = pl.pallas_call(
    _final_body,
    out_shape=jax.ShapeDtypeStruct((G, T), f32),
)


def kernel(dag_x, dag_edge_index, dag_layers_mask, dag_edge_attr, dag_readouts,
           dag_leaves, batch, Wt, bt, W1, b1, W2, b2, Wfc1, bfc1, Wfc2, bfc2):
    xpad = jnp.pad(dag_x, ((0, NP - N), (0, 0)))
    src2 = dag_edge_index[0].astype(i32).reshape(ER, EB)
    dst2 = dag_edge_index[1].astype(i32).reshape(ER, EB)
    msk2 = dag_layers_mask.astype(i32).reshape(ER, EB)
    att2 = dag_edge_attr.astype(f32).reshape(ER, EB)
    batchp = jnp.pad(batch.astype(i32), (0, NP - N))
    r2 = dag_readouts.astype(i32).reshape(RR, EB)
    leaves0 = dag_leaves[0].astype(i32)

    im = _sc_init()(leaves0)
    fL, fR, xL, xR = _tc_prep(xpad, Wt, bt.reshape(1, D), im.reshape(NP, 1))

    for i in range(L):
        li = jnp.full((16,), i, i32)
        exL, exR, cnt = _sc_layer()(xL, xR, src2, dst2, msk2, att2, li)
        xL, xR = _tc_layer(cnt.reshape(NP, 1), fL, fR, exL, exR, xL, xR,
                           W1[i], b1[i].reshape(1, D), W2[i], b2[i].reshape(1, D))

    oL, oR = _sc_read()(xL, xR, r2, batchp)
    xcat = jnp.concatenate(
        [jnp.concatenate([oL[i * G:(i + 1) * G], oR[i * G:(i + 1) * G]], axis=1)
         for i in range(L + 1)], axis=1)
    out = _tc_final(xcat, Wfc1, bfc1.reshape(1, D), Wfc2, bfc2.reshape(1, T))
    return out


# speedup vs baseline: 7.4988x; 2.0257x over previous
import functools

import jax
import jax.numpy as jnp
from jax import lax
from jax.experimental import pallas as pl
from jax.experimental.pallas import tpu as pltpu
from jax.experimental.pallas import tpu_sc as plsc

N, E, D, L, G, T, R = 10000, 160000, 256, 4, 64, 128, 2048
NP = 10240
H = D // 2
BN = 512
EB = 80
ER = E // EB
NT = 16
TPT = ER // NT
OCH = 25
NOU = TPT // OCH
STR = NP // NT
RR = (L + 1) * R // EB
RPT = RR // NT
OS = (L + 1) * G
PB = 2 * EB

f32 = jnp.float32
i32 = jnp.int32


@functools.cache
def _mesh():
    return plsc.VectorSubcoreMesh(core_axis_name="c", subcore_axis_name="s")


def _sc_init_body(lv_h, im_out, mv, lv):
    c = lax.axis_index("c")
    s = lax.axis_index("s")

    @pl.when(jnp.logical_and(c == 0, s == 0))
    def _():
        def _z(k, carry):
            mv[pl.ds(k * 16, 16)] = jnp.zeros((16,), f32)
            return carry
        lax.fori_loop(0, NP // 16, _z, None)
        pltpu.sync_copy(lv_h, lv)

        def _sc(g, carry):
            r16 = lv[pl.ds(g * 16, 16)]
            plsc.store_scatter(mv, [r16], jnp.ones((16,), f32))
            return carry
        lax.fori_loop(0, R // 16, _sc, None)
        pltpu.sync_copy(mv, im_out)


@functools.cache
def _sc_init():
    return pl.kernel(
        _sc_init_body,
        mesh=_mesh(),
        compiler_params=pltpu.CompilerParams(needs_layout_passes=False, use_tc_tiling_on_sc=False),
        out_type=jax.ShapeDtypeStruct((NP,), f32),
        scratch_types=[
            pltpu.VMEM((NP,), f32),
            pltpu.VMEM((R,), i32),
        ],
    )


def _sc_layer_body(xL, xR, src2, dst2, msk2, att2, li_h,
                   exL, exR, cnt,
                   srcv, dstv, mskv, attv, srcp, dstp, attp, dstf, onesv,
                   rows, liv, exsh, cntsh, sem):
    c = lax.axis_index("c")
    s = lax.axis_index("s")
    pltpu.sync_copy(li_h, liv)
    lid = liv[pl.ds(0, 16)][0]

    def _zrow(k, carry):
        rows[k // 8, pl.ds((k % 8) * 16, 16)] = jnp.zeros((16,), f32)
        return carry
    lax.fori_loop(0, EB * 8, _zrow, None)
    for k in range(STR // EB):
        pltpu.sync_copy(rows, exsh.at[pl.ds(s * STR + k * EB, EB)])
    for g in range(EB // 16):
        onesv[pl.ds(g * 16, 16)] = jnp.zeros((16,), f32)
    for k in range(STR // EB):
        pltpu.sync_copy(onesv, cntsh.at[pl.ds(s * STR + k * EB, EB)])
    for k in range(PB // 16):
        ksl = pl.ds(k * 16, 16)
        srcp[ksl] = jnp.zeros((16,), i32)
        dstp[ksl] = jnp.zeros((16,), i32)
        attp[ksl] = jnp.full((16,), -1.0, f32)
    plsc.subcore_barrier()

    def _fire():
        for k in range(EB // 16):
            ksl = pl.ds(k * 16, 16)
            dstf[ksl] = dstp[ksl]

        @pl.when(c == 0)
        def _():
            pltpu.async_copy(xL.at[srcp.at[pl.ds(0, EB)]], rows, sem).wait()

        @pl.when(c == 1)
        def _():
            pltpu.async_copy(xR.at[srcp.at[pl.ds(0, EB)]], rows, sem).wait()

        for g in range(EB // 16):
            gsl = pl.ds(g * 16, 16)
            a16 = attp[gsl]
            w16 = jnp.maximum(a16, 0.0)
            onesv[gsl] = jnp.where(a16 >= 0.0, 1.0, 0.0)
            for l in range(16):
                w = w16[l]
                e = g * 16 + l
                for cb in range(8):
                    sl = pl.ds(cb * 16, 16)
                    rows[e, sl] = rows[e, sl] * w

        @pl.when(c == 0)
        def _():
            pltpu.sync_copy(onesv, cntsh.at[dstf], add=True)

        pltpu.sync_copy(rows, exsh.at[dstf], add=True)

    def _outer(o, npend):
        rowbase = s * TPT + o * OCH
        pltpu.sync_copy(src2.at[pl.ds(rowbase, OCH)], srcv)
        pltpu.sync_copy(dst2.at[pl.ds(rowbase, OCH)], dstv)
        pltpu.sync_copy(msk2.at[pl.ds(rowbase, OCH)], mskv)
        pltpu.sync_copy(att2.at[pl.ds(rowbase, OCH)], attv)

        def _grp(g, np_in):
            row = g // (EB // 16)
            col = (g % (EB // 16)) * 16
            gsl = pl.ds(col, 16)
            m16 = mskv[row, gsl] == lid
            plsc.store_compressed(srcp.at[pl.ds(np_in, 16)], srcv[row, gsl], mask=m16)
            plsc.store_compressed(dstp.at[pl.ds(np_in, 16)], dstv[row, gsl], mask=m16)
            plsc.store_compressed(attp.at[pl.ds(np_in, 16)], attv[row, gsl], mask=m16)
            np2 = np_in + plsc.all_reduce_population_count(m16)[0]

            @pl.when(np2 >= EB)
            def _():
                _fire()
                s16 = srcp[pl.ds(EB, 16)]
                d16 = dstp[pl.ds(EB, 16)]
                a16 = attp[pl.ds(EB, 16)]
                srcp[pl.ds(0, 16)] = s16
                dstp[pl.ds(0, 16)] = d16
                attp[pl.ds(0, 16)] = a16
            return np2 - jnp.where(np2 >= EB, EB, 0)
        return lax.fori_loop(0, OCH * (EB // 16), _grp, npend)
    npend = lax.fori_loop(0, NOU, _outer, jnp.int32(0))

    @pl.when(npend > 0)
    def _():
        for k in range(EB // 16):
            attp[pl.ds(npend + k * 16, 16)] = jnp.full((16,), -1.0, f32)
        _fire()
    plsc.subcore_barrier()

    @pl.when(c == 0)
    def _():
        pltpu.sync_copy(exsh.at[pl.ds(s * STR, STR)], exL.at[pl.ds(s * STR, STR)])
        pltpu.sync_copy(cntsh.at[pl.ds(s * STR, STR)], cnt.at[pl.ds(s * STR, STR)])

    @pl.when(c == 1)
    def _():
        pltpu.sync_copy(exsh.at[pl.ds(s * STR, STR)], exR.at[pl.ds(s * STR, STR)])


@functools.cache
def _sc_layer():
  return pl.kernel(
    _sc_layer_body,
    mesh=_mesh(),
    compiler_params=pltpu.CompilerParams(needs_layout_passes=False, use_tc_tiling_on_sc=False),
    out_type=[
        jax.ShapeDtypeStruct((NP, H), f32),
        jax.ShapeDtypeStruct((NP, H), f32),
        jax.ShapeDtypeStruct((NP,), f32),
    ],
    scratch_types=[
        pltpu.VMEM((OCH, EB), i32),
        pltpu.VMEM((OCH, EB), i32),
        pltpu.VMEM((OCH, EB), i32),
        pltpu.VMEM((OCH, EB), f32),
        pltpu.VMEM((PB,), i32),
        pltpu.VMEM((PB,), i32),
        pltpu.VMEM((PB,), f32),
        pltpu.VMEM((EB,), i32),
        pltpu.VMEM((EB,), f32),
        pltpu.VMEM((EB, H), f32),
        pltpu.VMEM((16,), i32),
        pltpu.VMEM_SHARED((NP, H), f32),
        pltpu.VMEM_SHARED((NP,), f32),
        pltpu.SemaphoreType.DMA,
    ],
  )


def _sc_read_body(xL, xR, r2, batchp, oL, oR, rv, segv, rows, bv, osh, sem):
    c = lax.axis_index("c")
    s = lax.axis_index("s")
    pltpu.sync_copy(batchp, bv)
    pltpu.sync_copy(r2.at[pl.ds(s * RPT, RPT)], rv)

    def _zrow(k, carry):
        rows[k // 8, pl.ds((k % 8) * 16, 16)] = jnp.zeros((16,), f32)
        return carry
    lax.fori_loop(0, (OS // NT) * 8, _zrow, None)
    pltpu.sync_copy(rows.at[pl.ds(0, OS // NT)],
                    osh.at[pl.ds(s * (OS // NT), OS // NT)])
    plsc.subcore_barrier()

    for j in range(RPT):
        for g in range(EB // 16):
            sl = pl.ds(g * 16, 16)
            r16 = rv[j, sl]
            seg16 = plsc.load_gather(bv, [r16])
            flat = (s * RPT + j) * EB + g * 16
            segv[j, sl] = seg16 + (flat // R) * G

        @pl.when(c == 0)
        def _():
            pltpu.async_copy(xL.at[rv.at[j]], rows, sem).wait()

        @pl.when(c == 1)
        def _():
            pltpu.async_copy(xR.at[rv.at[j]], rows, sem).wait()

        pltpu.sync_copy(rows, osh.at[segv.at[j]], add=True)
    plsc.subcore_barrier()

    @pl.when(c == 0)
    def _():
        pltpu.sync_copy(osh.at[pl.ds(s * (OS // NT), OS // NT)],
                        oL.at[pl.ds(s * (OS // NT), OS // NT)])

    @pl.when(c == 1)
    def _():
        pltpu.sync_copy(osh.at[pl.ds(s * (OS // NT), OS // NT)],
                        oR.at[pl.ds(s * (OS // NT), OS // NT)])


@functools.cache
def _sc_read():
  return pl.kernel(
    _sc_read_body,
    mesh=_mesh(),
    compiler_params=pltpu.CompilerParams(needs_layout_passes=False, use_tc_tiling_on_sc=False),
    out_type=[
        jax.ShapeDtypeStruct((OS, H), f32),
        jax.ShapeDtypeStruct((OS, H), f32),
    ],
    scratch_types=[
        pltpu.VMEM((RPT, EB), i32),
        pltpu.VMEM((RPT, EB), i32),
        pltpu.VMEM((EB, H), f32),
        pltpu.VMEM((NP,), i32),
        pltpu.VMEM_SHARED((OS, H), f32),
        pltpu.SemaphoreType.DMA,
    ],
  )


def _prep_body(x_ref, w_ref, b_ref, m_ref, fL_ref, fR_ref, xL_ref, xR_ref):
    f = jnp.maximum(
        jnp.dot(x_ref[...], w_ref[...], preferred_element_type=f32) + b_ref[...],
        0.0)
    m = m_ref[...]
    fL_ref[...] = f[:, :H]
    fR_ref[...] = f[:, H:]
    xL_ref[...] = f[:, :H] * m
    xR_ref[...] = f[:, H:] * m


_tc_prep = pl.pallas_call(
    _prep_body,
    grid=(NP // BN,),
    in_specs=[
        pl.BlockSpec((BN, D), lambda i: (i, 0)),
        pl.BlockSpec((D, D), lambda i: (0, 0)),
        pl.BlockSpec((1, D), lambda i: (0, 0)),
        pl.BlockSpec((BN, 1), lambda i: (i, 0)),
    ],
    out_specs=[pl.BlockSpec((BN, H), lambda i: (i, 0))] * 4,
    out_shape=[jax.ShapeDtypeStruct((NP, H), f32)] * 4,
)


def _layer_body(cnt_ref, fL_ref, fR_ref, eL_ref, eR_ref, xL_ref, xR_ref,
                w1_ref, b1_ref, w2_ref, b2_ref, oL_ref, oR_ref):
    t = (cnt_ref[...] > 0.0).astype(f32)
    sL = eL_ref[...] + fL_ref[...] * t
    sR = eR_ref[...] + fR_ref[...] * t
    sfull = jnp.concatenate([sL, sR], axis=1)
    h = jnp.maximum(
        jnp.dot(sfull, w1_ref[...], preferred_element_type=f32) + b1_ref[...],
        0.0)
    h = jnp.maximum(
        jnp.dot(h, w2_ref[...], preferred_element_type=f32) + b2_ref[...],
        0.0)
    oL_ref[...] = xL_ref[...] + h[:, :H] * t
    oR_ref[...] = xR_ref[...] + h[:, H:] * t


_tc_layer = pl.pallas_call(
    _layer_body,
    grid=(NP // BN,),
    in_specs=[
        pl.BlockSpec((BN, 1), lambda i: (i, 0)),
        pl.BlockSpec((BN, H), lambda i: (i, 0)),
        pl.BlockSpec((BN, H), lambda i: (i, 0)),
        pl.BlockSpec((BN, H), lambda i: (i, 0)),
        pl.BlockSpec((BN, H), lambda i: (i, 0)),
        pl.BlockSpec((BN, H), lambda i: (i, 0)),
        pl.BlockSpec((BN, H), lambda i: (i, 0)),
        pl.BlockSpec((D, D), lambda i: (0, 0)),
        pl.BlockSpec((1, D), lambda i: (0, 0)),
        pl.BlockSpec((D, D), lambda i: (0, 0)),
        pl.BlockSpec((1, D), lambda i: (0, 0)),
    ],
    out_specs=[pl.BlockSpec((BN, H), lambda i: (i, 0))] * 2,
    out_shape=[jax.ShapeDtypeStruct((NP, H), f32)] * 2,
)


def _final_body(x_ref, w1_ref, b1_ref, w2_ref, b2_ref, o_ref):
    h = jnp.maximum(
        jnp.dot(x_ref[...], w1_ref[...], preferred_element_type=f32) + b1_ref[...],
        0.0)
    o_ref[...] = jnp.dot(h, w2_ref[...], preferred_element_type=f32) + b2_ref[...]


_tc_final = pl.pallas_call(
    _final_body,
    out_shape=jax.ShapeDtypeStruct((G, T), f32),
)


def kernel(dag_x, dag_edge_index, dag_layers_mask, dag_edge_attr, dag_readouts,
           dag_leaves, batch, Wt, bt, W1, b1, W2, b2, Wfc1, bfc1, Wfc2, bfc2):
    xpad = jnp.pad(dag_x, ((0, NP - N), (0, 0)))
    src2 = dag_edge_index[0].astype(i32).reshape(ER, EB)
    dst2 = dag_edge_index[1].astype(i32).reshape(ER, EB)
    msk2 = dag_layers_mask.astype(i32).reshape(ER, EB)
    att2 = dag_edge_attr.astype(f32).reshape(ER, EB)
    batchp = jnp.pad(batch.astype(i32), (0, NP - N))
    r2 = dag_readouts.astype(i32).reshape(RR, EB)
    leaves0 = dag_leaves[0].astype(i32)

    im = _sc_init()(leaves0)
    fL, fR, xL, xR = _tc_prep(xpad, Wt, bt.reshape(1, D), im.reshape(NP, 1))

    for i in range(L):
        li = jnp.full((16,), i, i32)
        exL, exR, cnt = _sc_layer()(xL, xR, src2, dst2, msk2, att2, li)
        xL, xR = _tc_layer(cnt.reshape(NP, 1), fL, fR, exL, exR, xL, xR,
                           W1[i], b1[i].reshape(1, D), W2[i], b2[i].reshape(1, D))

    oL, oR = _sc_read()(xL, xR, r2, batchp)
    xcat = jnp.concatenate(
        [jnp.concatenate([oL[i * G:(i + 1) * G], oR[i * G:(i + 1) * G]], axis=1)
         for i in range(L + 1)], axis=1)
    out = _tc_final(xcat, Wfc1, bfc1.reshape(1, D), Wfc2, bfc2.reshape(1, T))
    return out


# speedup vs baseline: 8.0660x; 1.0756x over previous
import functools

import jax
import jax.numpy as jnp
from jax import lax
from jax.experimental import pallas as pl
from jax.experimental.pallas import tpu as pltpu
from jax.experimental.pallas import tpu_sc as plsc

N, E, D, L, G, T, R = 10000, 160000, 256, 4, 64, 128, 2048
NP = 10240
H = D // 2
BN = 512
EB = 80
ER = E // EB
NT = 16
TPT = ER // NT
OCH = 25
NOU = TPT // OCH
STR = NP // NT
RR = (L + 1) * R // EB
RPT = RR // NT
OS = (L + 1) * G
PB = 2 * EB

f32 = jnp.float32
i32 = jnp.int32


@functools.cache
def _mesh():
    return plsc.VectorSubcoreMesh(core_axis_name="c", subcore_axis_name="s")


def _sc_init_body(lv_h, im_out, mv, lv):
    c = lax.axis_index("c")
    s = lax.axis_index("s")

    @pl.when(jnp.logical_and(c == 0, s == 0))
    def _():
        def _z(k, carry):
            mv[pl.ds(k * 16, 16)] = jnp.zeros((16,), f32)
            return carry
        lax.fori_loop(0, NP // 16, _z, None)
        pltpu.sync_copy(lv_h, lv)

        def _sc(g, carry):
            r16 = lv[pl.ds(g * 16, 16)]
            plsc.store_scatter(mv, [r16], jnp.ones((16,), f32))
            return carry
        lax.fori_loop(0, R // 16, _sc, None)
        pltpu.sync_copy(mv, im_out)


@functools.cache
def _sc_init():
    return pl.kernel(
        _sc_init_body,
        mesh=_mesh(),
        compiler_params=pltpu.CompilerParams(needs_layout_passes=False, use_tc_tiling_on_sc=False),
        out_type=jax.ShapeDtypeStruct((NP,), f32),
        scratch_types=[
            pltpu.VMEM((NP,), f32),
            pltpu.VMEM((R,), i32),
        ],
    )


def _sc_layer_body(xL, xR, src2, dst2, msk2, att2, li_h,
                   exL, exR, cnt,
                   srcv, dstv, mskv, attv, srcp, dstp, attp,
                   srcf0, dstf0, attf0, ones0, rows0,
                   srcf1, dstf1, attf1, ones1, rows1,
                   liv, exsh, cntsh, sem0, sem1):
    c = lax.axis_index("c")
    s = lax.axis_index("s")
    pltpu.sync_copy(li_h, liv)
    lid = liv[pl.ds(0, 16)][0]

    def _zrow(k, carry):
        rows0[k // 8, pl.ds((k % 8) * 16, 16)] = jnp.zeros((16,), f32)
        return carry
    lax.fori_loop(0, EB * 8, _zrow, None)
    for k in range(STR // EB):
        pltpu.sync_copy(rows0, exsh.at[pl.ds(s * STR + k * EB, EB)])
    for g in range(EB // 16):
        ones0[pl.ds(g * 16, 16)] = jnp.zeros((16,), f32)
    for k in range(STR // EB):
        pltpu.sync_copy(ones0, cntsh.at[pl.ds(s * STR + k * EB, EB)])
    for k in range(PB // 16):
        ksl = pl.ds(k * 16, 16)
        srcp[ksl] = jnp.zeros((16,), i32)
        dstp[ksl] = jnp.zeros((16,), i32)
        attp[ksl] = jnp.full((16,), -1.0, f32)
    plsc.subcore_barrier()

    def _issue_slot(srcf, dstf, attf, rows, sem):
        for k in range(EB // 16):
            ksl = pl.ds(k * 16, 16)
            srcf[ksl] = srcp[ksl]
            dstf[ksl] = dstp[ksl]
            attf[ksl] = attp[ksl]

        @pl.when(c == 0)
        def _():
            pltpu.async_copy(xL.at[srcf.at[pl.ds(0, EB)]], rows, sem)

        @pl.when(c == 1)
        def _():
            pltpu.async_copy(xR.at[srcf.at[pl.ds(0, EB)]], rows, sem)

    def _process_slot(dstf, attf, ones, rows, sem):
        @pl.when(c == 0)
        def _():
            for g in range(EB // 16):
                gsl = pl.ds(g * 16, 16)
                a16 = attf[gsl]
                ones[gsl] = jnp.where(a16 >= 0.0, 1.0, 0.0)
            pltpu.sync_copy(ones, cntsh.at[dstf], add=True)

        pltpu.make_async_copy(xL.at[pl.ds(0, EB)], rows, sem).wait()
        for g in range(EB // 16):
            gsl = pl.ds(g * 16, 16)
            w16 = jnp.maximum(attf[gsl], 0.0)
            for l in range(16):
                w = w16[l]
                e = g * 16 + l
                for cb in range(8):
                    sl = pl.ds(cb * 16, 16)
                    rows[e, sl] = rows[e, sl] * w
        pltpu.sync_copy(rows, exsh.at[dstf], add=True)

    def _issue(k):
        @pl.when(k % 2 == 0)
        def _():
            _issue_slot(srcf0, dstf0, attf0, rows0, sem0)

        @pl.when(k % 2 == 1)
        def _():
            _issue_slot(srcf1, dstf1, attf1, rows1, sem1)

    def _process(k):
        @pl.when(k % 2 == 0)
        def _():
            _process_slot(dstf0, attf0, ones0, rows0, sem0)

        @pl.when(k % 2 == 1)
        def _():
            _process_slot(dstf1, attf1, ones1, rows1, sem1)

    def _outer(o, carry):
        rowbase = s * TPT + o * OCH
        pltpu.sync_copy(src2.at[pl.ds(rowbase, OCH)], srcv)
        pltpu.sync_copy(dst2.at[pl.ds(rowbase, OCH)], dstv)
        pltpu.sync_copy(msk2.at[pl.ds(rowbase, OCH)], mskv)
        pltpu.sync_copy(att2.at[pl.ds(rowbase, OCH)], attv)

        def _grp(g, carry_in):
            np_in, nf = carry_in
            row = g // (EB // 16)
            col = (g % (EB // 16)) * 16
            gsl = pl.ds(col, 16)
            m16 = mskv[row, gsl] == lid
            plsc.store_compressed(srcp.at[pl.ds(np_in, 16)], srcv[row, gsl], mask=m16)
            plsc.store_compressed(dstp.at[pl.ds(np_in, 16)], dstv[row, gsl], mask=m16)
            plsc.store_compressed(attp.at[pl.ds(np_in, 16)], attv[row, gsl], mask=m16)
            np2 = np_in + plsc.all_reduce_population_count(m16)[0]

            @pl.when(np2 >= EB)
            def _():
                _issue(nf)

                @pl.when(nf > 0)
                def _():
                    _process(nf - 1)
                s16 = srcp[pl.ds(EB, 16)]
                d16 = dstp[pl.ds(EB, 16)]
                a16 = attp[pl.ds(EB, 16)]
                srcp[pl.ds(0, 16)] = s16
                dstp[pl.ds(0, 16)] = d16
                attp[pl.ds(0, 16)] = a16
            fired = np2 >= EB
            return (np2 - jnp.where(fired, EB, 0),
                    nf + jnp.where(fired, 1, 0))
        return lax.fori_loop(0, OCH * (EB // 16), _grp, carry)
    npend, nfire = lax.fori_loop(0, NOU, _outer, (jnp.int32(0), jnp.int32(0)))

    @pl.when(npend > 0)
    def _():
        for k in range(EB // 16):
            attp[pl.ds(npend + k * 16, 16)] = jnp.full((16,), -1.0, f32)
        _issue(nfire)

        @pl.when(nfire > 0)
        def _():
            _process(nfire - 1)
    nfire = nfire + jnp.where(npend > 0, 1, 0)

    @pl.when(nfire > 0)
    def _():
        _process(nfire - 1)
    plsc.subcore_barrier()

    @pl.when(c == 0)
    def _():
        pltpu.sync_copy(exsh.at[pl.ds(s * STR, STR)], exL.at[pl.ds(s * STR, STR)])
        pltpu.sync_copy(cntsh.at[pl.ds(s * STR, STR)], cnt.at[pl.ds(s * STR, STR)])

    @pl.when(c == 1)
    def _():
        pltpu.sync_copy(exsh.at[pl.ds(s * STR, STR)], exR.at[pl.ds(s * STR, STR)])


@functools.cache
def _sc_layer():
  return pl.kernel(
    _sc_layer_body,
    mesh=_mesh(),
    compiler_params=pltpu.CompilerParams(needs_layout_passes=False, use_tc_tiling_on_sc=False),
    out_type=[
        jax.ShapeDtypeStruct((NP, H), f32),
        jax.ShapeDtypeStruct((NP, H), f32),
        jax.ShapeDtypeStruct((NP,), f32),
    ],
    scratch_types=[
        pltpu.VMEM((OCH, EB), i32),
        pltpu.VMEM((OCH, EB), i32),
        pltpu.VMEM((OCH, EB), i32),
        pltpu.VMEM((OCH, EB), f32),
        pltpu.VMEM((PB,), i32),
        pltpu.VMEM((PB,), i32),
        pltpu.VMEM((PB,), f32),
        pltpu.VMEM((EB,), i32),
        pltpu.VMEM((EB,), i32),
        pltpu.VMEM((EB,), f32),
        pltpu.VMEM((EB,), f32),
        pltpu.VMEM((EB, H), f32),
        pltpu.VMEM((EB,), i32),
        pltpu.VMEM((EB,), i32),
        pltpu.VMEM((EB,), f32),
        pltpu.VMEM((EB,), f32),
        pltpu.VMEM((EB, H), f32),
        pltpu.VMEM((16,), i32),
        pltpu.VMEM_SHARED((NP, H), f32),
        pltpu.VMEM_SHARED((NP,), f32),
        pltpu.SemaphoreType.DMA,
        pltpu.SemaphoreType.DMA,
    ],
  )


def _sc_read_body(xL, xR, r2, batchp, oL, oR, rv, segv, rows, bv, osh, sem):
    c = lax.axis_index("c")
    s = lax.axis_index("s")
    pltpu.sync_copy(batchp, bv)
    pltpu.sync_copy(r2.at[pl.ds(s * RPT, RPT)], rv)

    def _zrow(k, carry):
        rows[k // 8, pl.ds((k % 8) * 16, 16)] = jnp.zeros((16,), f32)
        return carry
    lax.fori_loop(0, (OS // NT) * 8, _zrow, None)
    pltpu.sync_copy(rows.at[pl.ds(0, OS // NT)],
                    osh.at[pl.ds(s * (OS // NT), OS // NT)])
    plsc.subcore_barrier()

    for j in range(RPT):
        for g in range(EB // 16):
            sl = pl.ds(g * 16, 16)
            r16 = rv[j, sl]
            seg16 = plsc.load_gather(bv, [r16])
            flat = (s * RPT + j) * EB + g * 16
            segv[j, sl] = seg16 + (flat // R) * G

        @pl.when(c == 0)
        def _():
            pltpu.async_copy(xL.at[rv.at[j]], rows, sem).wait()

        @pl.when(c == 1)
        def _():
            pltpu.async_copy(xR.at[rv.at[j]], rows, sem).wait()

        pltpu.sync_copy(rows, osh.at[segv.at[j]], add=True)
    plsc.subcore_barrier()

    @pl.when(c == 0)
    def _():
        pltpu.sync_copy(osh.at[pl.ds(s * (OS // NT), OS // NT)],
                        oL.at[pl.ds(s * (OS // NT), OS // NT)])

    @pl.when(c == 1)
    def _():
        pltpu.sync_copy(osh.at[pl.ds(s * (OS // NT), OS // NT)],
                        oR.at[pl.ds(s * (OS // NT), OS // NT)])


@functools.cache
def _sc_read():
  return pl.kernel(
    _sc_read_body,
    mesh=_mesh(),
    compiler_params=pltpu.CompilerParams(needs_layout_passes=False, use_tc_tiling_on_sc=False),
    out_type=[
        jax.ShapeDtypeStruct((OS, H), f32),
        jax.ShapeDtypeStruct((OS, H), f32),
    ],
    scratch_types=[
        pltpu.VMEM((RPT, EB), i32),
        pltpu.VMEM((RPT, EB), i32),
        pltpu.VMEM((EB, H), f32),
        pltpu.VMEM((NP,), i32),
        pltpu.VMEM_SHARED((OS, H), f32),
        pltpu.SemaphoreType.DMA,
    ],
  )


def _prep_body(x_ref, w_ref, b_ref, m_ref, fL_ref, fR_ref, xL_ref, xR_ref):
    f = jnp.maximum(
        jnp.dot(x_ref[...], w_ref[...], preferred_element_type=f32) + b_ref[...],
        0.0)
    m = m_ref[...]
    fL_ref[...] = f[:, :H]
    fR_ref[...] = f[:, H:]
    xL_ref[...] = f[:, :H] * m
    xR_ref[...] = f[:, H:] * m


_tc_prep = pl.pallas_call(
    _prep_body,
    grid=(NP // BN,),
    in_specs=[
        pl.BlockSpec((BN, D), lambda i: (i, 0)),
        pl.BlockSpec((D, D), lambda i: (0, 0)),
        pl.BlockSpec((1, D), lambda i: (0, 0)),
        pl.BlockSpec((BN, 1), lambda i: (i, 0)),
    ],
    out_specs=[pl.BlockSpec((BN, H), lambda i: (i, 0))] * 4,
    out_shape=[jax.ShapeDtypeStruct((NP, H), f32)] * 4,
)


def _layer_body(cnt_ref, fL_ref, fR_ref, eL_ref, eR_ref, xL_ref, xR_ref,
                w1_ref, b1_ref, w2_ref, b2_ref, oL_ref, oR_ref):
    t = (cnt_ref[...] > 0.0).astype(f32)
    sL = eL_ref[...] + fL_ref[...] * t
    sR = eR_ref[...] + fR_ref[...] * t
    sfull = jnp.concatenate([sL, sR], axis=1)
    h = jnp.maximum(
        jnp.dot(sfull, w1_ref[...], preferred_element_type=f32) + b1_ref[...],
        0.0)
    h = jnp.maximum(
        jnp.dot(h, w2_ref[...], preferred_element_type=f32) + b2_ref[...],
        0.0)
    oL_ref[...] = xL_ref[...] + h[:, :H] * t
    oR_ref[...] = xR_ref[...] + h[:, H:] * t


_tc_layer = pl.pallas_call(
    _layer_body,
    grid=(NP // BN,),
    in_specs=[
        pl.BlockSpec((BN, 1), lambda i: (i, 0)),
        pl.BlockSpec((BN, H), lambda i: (i, 0)),
        pl.BlockSpec((BN, H), lambda i: (i, 0)),
        pl.BlockSpec((BN, H), lambda i: (i, 0)),
        pl.BlockSpec((BN, H), lambda i: (i, 0)),
        pl.BlockSpec((BN, H), lambda i: (i, 0)),
        pl.BlockSpec((BN, H), lambda i: (i, 0)),
        pl.BlockSpec((D, D), lambda i: (0, 0)),
        pl.BlockSpec((1, D), lambda i: (0, 0)),
        pl.BlockSpec((D, D), lambda i: (0, 0)),
        pl.BlockSpec((1, D), lambda i: (0, 0)),
    ],
    out_specs=[pl.BlockSpec((BN, H), lambda i: (i, 0))] * 2,
    out_shape=[jax.ShapeDtypeStruct((NP, H), f32)] * 2,
)


def _final_body(x_ref, w1_ref, b1_ref, w2_ref, b2_ref, o_ref):
    h = jnp.maximum(
        jnp.dot(x_ref[...], w1_ref[...], preferred_element_type=f32) + b1_ref[...],
        0.0)
    o_ref[...] = jnp.dot(h, w2_ref[...], preferred_element_type=f32) + b2_ref[...]


_tc_final = pl.pallas_call(
    _final_body,
    out_shape=jax.ShapeDtypeStruct((G, T), f32),
)


def kernel(dag_x, dag_edge_index, dag_layers_mask, dag_edge_attr, dag_readouts,
           dag_leaves, batch, Wt, bt, W1, b1, W2, b2, Wfc1, bfc1, Wfc2, bfc2):
    xpad = jnp.pad(dag_x, ((0, NP - N), (0, 0)))
    src2 = dag_edge_index[0].astype(i32).reshape(ER, EB)
    dst2 = dag_edge_index[1].astype(i32).reshape(ER, EB)
    msk2 = dag_layers_mask.astype(i32).reshape(ER, EB)
    att2 = dag_edge_attr.astype(f32).reshape(ER, EB)
    batchp = jnp.pad(batch.astype(i32), (0, NP - N))
    r2 = dag_readouts.astype(i32).reshape(RR, EB)
    leaves0 = dag_leaves[0].astype(i32)

    im = _sc_init()(leaves0)
    fL, fR, xL, xR = _tc_prep(xpad, Wt, bt.reshape(1, D), im.reshape(NP, 1))

    for i in range(L):
        li = jnp.full((16,), i, i32)
        exL, exR, cnt = _sc_layer()(xL, xR, src2, dst2, msk2, att2, li)
        xL, xR = _tc_layer(cnt.reshape(NP, 1), fL, fR, exL, exR, xL, xR,
                           W1[i], b1[i].reshape(1, D), W2[i], b2[i].reshape(1, D))

    oL, oR = _sc_read()(xL, xR, r2, batchp)
    xcat = jnp.concatenate(
        [jnp.concatenate([oL[i * G:(i + 1) * G], oR[i * G:(i + 1) * G]], axis=1)
         for i in range(L + 1)], axis=1)
    out = _tc_final(xcat, Wfc1, bfc1.reshape(1, D), Wfc2, bfc2.reshape(1, T))
    return out


# speedup vs baseline: 8.1303x; 1.0080x over previous
import functools

import jax
import jax.numpy as jnp
from jax import lax
from jax.experimental import pallas as pl
from jax.experimental.pallas import tpu as pltpu
from jax.experimental.pallas import tpu_sc as plsc

N, E, D, L, G, T, R = 10000, 160000, 256, 4, 64, 128, 2048
NP = 10240
H = D // 2
BN = 512
EB = 80
ER = E // EB
NT = 16
TPT = ER // NT
OCH = 25
NOU = TPT // OCH
STR = NP // NT
RR = (L + 1) * R // EB
RPT = RR // NT
OS = (L + 1) * G
PB = 2 * EB
FB = 400
PB2 = FB + EB
CAP = TPT * EB + EB
PTOT = L * NT * CAP
LR = L * NP // H
LRT = LR // NT

f32 = jnp.float32
i32 = jnp.int32


@functools.cache
def _mesh():
    return plsc.VectorSubcoreMesh(core_axis_name="c", subcore_axis_name="s")


def _sc_prep_body(src2, dst2, msk2, att2, lv_h,
                  im_out, lcnt, psrc, pdst, patt, pcnt,
                  srcv, dstv, mskv, attv,
                  pps0, pps1, pps2, pps3,
                  ppd0, ppd1, ppd2, ppd3,
                  ppa0, ppa1, ppa2, ppa3,
                  ss0, sd0, sa0, ss1, sd1, sa1,
                  cbuf, zv, idv, mv, lv, pcv, lcsh, semp0, semp1):
    c = lax.axis_index("c")
    s = lax.axis_index("s")
    iota16 = lax.iota(i32, 16)
    ppss = [pps0, pps1, pps2, pps3]
    ppds = [ppd0, ppd1, ppd2, ppd3]
    ppas = [ppa0, ppa1, ppa2, ppa3]

    @pl.when(c == 0)
    def _():
        for l in range(L):
            for k in range(PB2 // 16):
                ksl = pl.ds(k * 16, 16)
                ppss[l][ksl] = jnp.zeros((16,), i32)
                ppds[l][ksl] = jnp.zeros((16,), i32)

        def _drain_slot0():
            pltpu.make_async_copy(psrc.at[pl.ds(0, FB)], ss0, semp0).wait()
            pltpu.make_async_copy(pdst.at[pl.ds(0, FB)], sd0, semp0).wait()
            pltpu.make_async_copy(patt.at[pl.ds(0, FB)], sa0, semp0).wait()

        def _drain_slot1():
            pltpu.make_async_copy(psrc.at[pl.ds(0, FB)], ss1, semp1).wait()
            pltpu.make_async_copy(pdst.at[pl.ds(0, FB)], sd1, semp1).wait()
            pltpu.make_async_copy(patt.at[pl.ds(0, FB)], sa1, semp1).wait()

        def _flush(l, off, nfl):
            dst_base = pl.multiple_of((l * NT + s) * CAP + off, 8)

            @pl.when(nfl % 2 == 0)
            def _():
                @pl.when(nfl >= 2)
                def _():
                    _drain_slot0()
                for k in range(FB // 16):
                    ksl = pl.ds(k * 16, 16)
                    ss0[ksl] = ppss[l][ksl]
                    sd0[ksl] = ppds[l][ksl]
                    sa0[ksl] = ppas[l][ksl]
                pltpu.async_copy(ss0, psrc.at[pl.ds(dst_base, FB)], semp0)
                pltpu.async_copy(sd0, pdst.at[pl.ds(dst_base, FB)], semp0)
                pltpu.async_copy(sa0, patt.at[pl.ds(dst_base, FB)], semp0)

            @pl.when(nfl % 2 == 1)
            def _():
                @pl.when(nfl >= 2)
                def _():
                    _drain_slot1()
                for k in range(FB // 16):
                    ksl = pl.ds(k * 16, 16)
                    ss1[ksl] = ppss[l][ksl]
                    sd1[ksl] = ppds[l][ksl]
                    sa1[ksl] = ppas[l][ksl]
                pltpu.async_copy(ss1, psrc.at[pl.ds(dst_base, FB)], semp1)
                pltpu.async_copy(sd1, pdst.at[pl.ds(dst_base, FB)], semp1)
                pltpu.async_copy(sa1, patt.at[pl.ds(dst_base, FB)], semp1)

            sh16 = ppss[l][pl.ds(FB, 16)]
            dh16 = ppds[l][pl.ds(FB, 16)]
            ah16 = ppas[l][pl.ds(FB, 16)]
            ppss[l][pl.ds(0, 16)] = sh16
            ppds[l][pl.ds(0, 16)] = dh16
            ppas[l][pl.ds(0, 16)] = ah16

        def _outer(o, carry):
            rowbase = s * TPT + o * OCH
            pltpu.sync_copy(src2.at[pl.ds(rowbase, OCH)], srcv)
            pltpu.sync_copy(dst2.at[pl.ds(rowbase, OCH)], dstv)
            pltpu.sync_copy(msk2.at[pl.ds(rowbase, OCH)], mskv)
            pltpu.sync_copy(att2.at[pl.ds(rowbase, OCH)], attv)

            def _grp(g, carry_in):
                nps = list(carry_in[0:L])
                offs = list(carry_in[L:2 * L])
                nfl = carry_in[2 * L]
                row = g // (EB // 16)
                col = (g % (EB // 16)) * 16
                gsl = pl.ds(col, 16)
                s16 = srcv[row, gsl]
                d16 = dstv[row, gsl]
                k16 = mskv[row, gsl]
                a16 = attv[row, gsl]
                for l in range(L):
                    m16 = k16 == l
                    plsc.store_compressed(ppss[l].at[pl.ds(nps[l], 16)], s16, mask=m16)
                    plsc.store_compressed(ppds[l].at[pl.ds(nps[l], 16)], d16, mask=m16)
                    plsc.store_compressed(ppas[l].at[pl.ds(nps[l], 16)], a16, mask=m16)
                    npl = nps[l] + plsc.all_reduce_population_count(m16)[0]
                    fired = npl >= FB

                    @pl.when(fired)
                    def _(l=l, npl=npl, nfl=nfl):
                        _flush(l, offs[l], nfl)
                    nps[l] = npl - jnp.where(fired, FB, 0)
                    offs[l] = offs[l] + jnp.where(fired, FB, 0)
                    nfl = nfl + jnp.where(fired, 1, 0)
                return tuple(nps) + tuple(offs) + (nfl,)
            return lax.fori_loop(0, OCH * (EB // 16), _grp, carry)

        zero = jnp.int32(0)
        carry = lax.fori_loop(0, NOU, _outer, (zero,) * (2 * L + 1))
        nps = carry[0:L]
        offs = carry[L:2 * L]
        nfl = carry[2 * L]

        @pl.when(nfl >= 1)
        def _():
            @pl.when(nfl % 2 == 1)
            def _():
                _drain_slot0()

            @pl.when(nfl % 2 == 0)
            def _():
                _drain_slot1()

        @pl.when(nfl >= 2)
        def _():
            @pl.when(nfl % 2 == 0)
            def _():
                _drain_slot0()

            @pl.when(nfl % 2 == 1)
            def _():
                _drain_slot1()

        lane0 = iota16 == 0
        for l in range(L):
            npl16 = jnp.zeros((16,), i32) + nps[l]
            for k in range(PB2 // 16):
                ksl = pl.ds(k * 16, 16)
                cur = ppas[l][ksl]
                ppas[l][ksl] = jnp.where(iota16 + k * 16 >= npl16, -1.0, cur)
            tbase = pl.multiple_of((l * NT + s) * CAP + offs[l], 8)
            pltpu.sync_copy(ppss[l], psrc.at[pl.ds(tbase, PB2)])
            pltpu.sync_copy(ppds[l], pdst.at[pl.ds(tbase, PB2)])
            pltpu.sync_copy(ppas[l], patt.at[pl.ds(tbase, PB2)])
            cnt16 = jnp.zeros((16,), i32) + (offs[l] + nps[l])
            plsc.store_scatter(pcv, [jnp.full((16,), l, i32)], cnt16, mask=lane0)
        pltpu.sync_copy(pcv, pcnt.at[pl.ds(pl.multiple_of(s * 16, 8), 16)])

    @pl.when(c == 1)
    def _():
        def _zc(k, carry):
            cbuf[k // 8, pl.ds((k % 8) * 16, 16)] = jnp.zeros((16,), f32)
            return carry
        lax.fori_loop(0, (LR * 128) // 16, _zc, None)

        def _zz(k, carry):
            zv[k // 8, pl.ds((k % 8) * 16, 16)] = jnp.zeros((16,), f32)
            return carry
        lax.fori_loop(0, (LRT * 128) // 16, _zz, None)
        pltpu.sync_copy(zv, lcsh.at[pl.ds(s * LRT, LRT)])
        def _zi(k, carry):
            idv[pl.ds(k * 16, 16)] = iota16 + k * 16
            return carry
        lax.fori_loop(0, LR // 16, _zi, None)

    plsc.subcore_barrier()

    @pl.when(c == 1)
    def _():
        ones16 = jnp.ones((16,), f32)

        def _outer1(o, carry):
            rowbase = s * TPT + o * OCH
            pltpu.sync_copy(dst2.at[pl.ds(rowbase, OCH)], dstv)
            pltpu.sync_copy(msk2.at[pl.ds(rowbase, OCH)], mskv)

            def _grp1(g, carry_in):
                row = g // (EB // 16)
                col = (g % (EB // 16)) * 16
                gsl = pl.ds(col, 16)
                d16 = dstv[row, gsl]
                k16 = mskv[row, gsl]
                for l in range(L):
                    v16 = d16 + l * NP
                    plsc.addupdate_scatter(cbuf, [v16 >> 7, v16 & 127],
                                           ones16, mask=k16 == l)
                return carry_in
            return lax.fori_loop(0, OCH * (EB // 16), _grp1, carry)
        lax.fori_loop(0, NOU, _outer1, None)
        pltpu.sync_copy(cbuf, lcsh.at[idv], add=True)

        @pl.when(s == 0)
        def _():
            def _z(k, carry):
                mv[pl.ds(k * 16, 16)] = jnp.zeros((16,), f32)
                return carry
            lax.fori_loop(0, NP // 16, _z, None)
            pltpu.sync_copy(lv_h, lv)

            def _sc(g, carry):
                r16 = lv[pl.ds(g * 16, 16)]
                plsc.store_scatter(mv, [r16], jnp.ones((16,), f32))
                return carry
            lax.fori_loop(0, R // 16, _sc, None)
            pltpu.sync_copy(mv, im_out)

    plsc.subcore_barrier()

    @pl.when(c == 1)
    def _():
        pltpu.sync_copy(lcsh.at[pl.ds(s * LRT, LRT)], lcnt.at[pl.ds(s * LRT, LRT)])


@functools.cache
def _sc_prep():
    return pl.kernel(
        _sc_prep_body,
        mesh=_mesh(),
        compiler_params=pltpu.CompilerParams(needs_layout_passes=False, use_tc_tiling_on_sc=False),
        out_type=[
            jax.ShapeDtypeStruct((NP,), f32),
            jax.ShapeDtypeStruct((LR, H), f32),
            jax.ShapeDtypeStruct((PTOT,), i32),
            jax.ShapeDtypeStruct((PTOT,), i32),
            jax.ShapeDtypeStruct((PTOT,), f32),
            jax.ShapeDtypeStruct((NT * 16,), i32),
        ],
        scratch_types=[
            pltpu.VMEM((OCH, EB), i32),
            pltpu.VMEM((OCH, EB), i32),
            pltpu.VMEM((OCH, EB), i32),
            pltpu.VMEM((OCH, EB), f32),
            pltpu.VMEM((PB2,), i32),
            pltpu.VMEM((PB2,), i32),
            pltpu.VMEM((PB2,), i32),
            pltpu.VMEM((PB2,), i32),
            pltpu.VMEM((PB2,), i32),
            pltpu.VMEM((PB2,), i32),
            pltpu.VMEM((PB2,), i32),
            pltpu.VMEM((PB2,), i32),
            pltpu.VMEM((PB2,), f32),
            pltpu.VMEM((PB2,), f32),
            pltpu.VMEM((PB2,), f32),
            pltpu.VMEM((PB2,), f32),
            pltpu.VMEM((FB,), i32),
            pltpu.VMEM((FB,), i32),
            pltpu.VMEM((FB,), f32),
            pltpu.VMEM((FB,), i32),
            pltpu.VMEM((FB,), i32),
            pltpu.VMEM((FB,), f32),
            pltpu.VMEM((LR, H), f32),
            pltpu.VMEM((LRT, H), f32),
            pltpu.VMEM((LR,), i32),
            pltpu.VMEM((NP,), f32),
            pltpu.VMEM((R,), i32),
            pltpu.VMEM((16,), i32),
            pltpu.VMEM_SHARED((LR, H), f32),
            pltpu.SemaphoreType.DMA,
            pltpu.SemaphoreType.DMA,
        ],
    )


def _sc_layer_body(xL, xR, psrc, pdst, patt, pcnt, li_h,
                   exL, exR,
                   srcf0, dstf0, attf0, rows0,
                   srcf1, dstf1, attf1, rows1,
                   liv, pcv, exsh, sg0, sg1, ssc0, ssc1):
    c = lax.axis_index("c")
    s = lax.axis_index("s")
    pltpu.sync_copy(li_h, liv)
    lid = liv[pl.ds(0, 16)][0]
    pltpu.sync_copy(pcnt.at[pl.ds(pl.multiple_of(s * 16, 8), 16)], pcv)
    cnt = plsc.load_gather(pcv, [jnp.zeros((16,), i32) + lid])[0]
    nf = (cnt + (EB - 1)) // EB
    base = (lid * NT + s) * CAP

    def _zrow(k, carry):
        rows0[k // 8, pl.ds((k % 8) * 16, 16)] = jnp.zeros((16,), f32)
        return carry
    lax.fori_loop(0, EB * 8, _zrow, None)
    for k in range(STR // EB):
        pltpu.sync_copy(rows0, exsh.at[pl.ds(s * STR + k * EB, EB)])
    plsc.subcore_barrier()

    def _issue_slot(srcf, dstf, attf, rows, sg, ssc, k):
        @pl.when(k >= 2)
        def _():
            pltpu.make_async_copy(xL.at[pl.ds(0, EB)], rows, ssc).wait()
        blk = pl.ds(pl.multiple_of(base + k * EB, 8), EB)
        pltpu.sync_copy(psrc.at[blk], srcf)
        pltpu.sync_copy(pdst.at[blk], dstf)
        pltpu.sync_copy(patt.at[blk], attf)

        @pl.when(c == 0)
        def _():
            pltpu.async_copy(xL.at[srcf], rows, sg)

        @pl.when(c == 1)
        def _():
            pltpu.async_copy(xR.at[srcf], rows, sg)

    def _process_slot(dstf, attf, rows, sg, ssc):
        pltpu.make_async_copy(xL.at[pl.ds(0, EB)], rows, sg).wait()
        for g in range(EB // 16):
            gsl = pl.ds(g * 16, 16)
            w16 = jnp.maximum(attf[gsl], 0.0)
            for l in range(16):
                w = w16[l]
                e = g * 16 + l
                for cb in range(8):
                    sl = pl.ds(cb * 16, 16)
                    rows[e, sl] = rows[e, sl] * w
        pltpu.async_copy(rows, exsh.at[dstf], ssc, add=True)

    def _issue(k):
        @pl.when(k % 2 == 0)
        def _():
            _issue_slot(srcf0, dstf0, attf0, rows0, sg0, ssc0, k)

        @pl.when(k % 2 == 1)
        def _():
            _issue_slot(srcf1, dstf1, attf1, rows1, sg1, ssc1, k)

    def _process(k):
        @pl.when(k % 2 == 0)
        def _():
            _process_slot(dstf0, attf0, rows0, sg0, ssc0)

        @pl.when(k % 2 == 1)
        def _():
            _process_slot(dstf1, attf1, rows1, sg1, ssc1)

    @pl.when(nf > 0)
    def _():
        _issue(0)

    def _pipe(k, carry):
        _issue(k)
        _process(k - 1)
        return carry
    lax.fori_loop(1, nf, _pipe, None)

    @pl.when(nf > 0)
    def _():
        _process(nf - 1)

    @pl.when(nf >= 1)
    def _():
        @pl.when((nf - 1) % 2 == 0)
        def _():
            pltpu.make_async_copy(xL.at[pl.ds(0, EB)], rows0, ssc0).wait()

        @pl.when((nf - 1) % 2 == 1)
        def _():
            pltpu.make_async_copy(xL.at[pl.ds(0, EB)], rows1, ssc1).wait()

    @pl.when(nf >= 2)
    def _():
        @pl.when(nf % 2 == 0)
        def _():
            pltpu.make_async_copy(xL.at[pl.ds(0, EB)], rows0, ssc0).wait()

        @pl.when(nf % 2 == 1)
        def _():
            pltpu.make_async_copy(xL.at[pl.ds(0, EB)], rows1, ssc1).wait()
    plsc.subcore_barrier()

    @pl.when(c == 0)
    def _():
        pltpu.sync_copy(exsh.at[pl.ds(s * STR, STR)], exL.at[pl.ds(s * STR, STR)])

    @pl.when(c == 1)
    def _():
        pltpu.sync_copy(exsh.at[pl.ds(s * STR, STR)], exR.at[pl.ds(s * STR, STR)])


@functools.cache
def _sc_layer():
  return pl.kernel(
    _sc_layer_body,
    mesh=_mesh(),
    compiler_params=pltpu.CompilerParams(needs_layout_passes=False, use_tc_tiling_on_sc=False),
    out_type=[
        jax.ShapeDtypeStruct((NP, H), f32),
        jax.ShapeDtypeStruct((NP, H), f32),
    ],
    scratch_types=[
        pltpu.VMEM((EB,), i32),
        pltpu.VMEM((EB,), i32),
        pltpu.VMEM((EB,), f32),
        pltpu.VMEM((EB, H), f32),
        pltpu.VMEM((EB,), i32),
        pltpu.VMEM((EB,), i32),
        pltpu.VMEM((EB,), f32),
        pltpu.VMEM((EB, H), f32),
        pltpu.VMEM((16,), i32),
        pltpu.VMEM((16,), i32),
        pltpu.VMEM_SHARED((NP, H), f32),
        pltpu.SemaphoreType.DMA,
        pltpu.SemaphoreType.DMA,
        pltpu.SemaphoreType.DMA,
        pltpu.SemaphoreType.DMA,
    ],
  )


def _sc_read_body(xL, xR, r2, batchp, oL, oR, rv, segv, rows, bv, osh, sem):
    c = lax.axis_index("c")
    s = lax.axis_index("s")
    pltpu.sync_copy(batchp, bv)
    pltpu.sync_copy(r2.at[pl.ds(s * RPT, RPT)], rv)

    def _zrow(k, carry):
        rows[k // 8, pl.ds((k % 8) * 16, 16)] = jnp.zeros((16,), f32)
        return carry
    lax.fori_loop(0, (OS // NT) * 8, _zrow, None)
    pltpu.sync_copy(rows.at[pl.ds(0, OS // NT)],
                    osh.at[pl.ds(s * (OS // NT), OS // NT)])
    plsc.subcore_barrier()

    for j in range(RPT):
        for g in range(EB // 16):
            sl = pl.ds(g * 16, 16)
            r16 = rv[j, sl]
            seg16 = plsc.load_gather(bv, [r16])
            flat = (s * RPT + j) * EB + g * 16
            segv[j, sl] = seg16 + (flat // R) * G

        @pl.when(c == 0)
        def _():
            pltpu.async_copy(xL.at[rv.at[j]], rows, sem).wait()

        @pl.when(c == 1)
        def _():
            pltpu.async_copy(xR.at[rv.at[j]], rows, sem).wait()

        pltpu.sync_copy(rows, osh.at[segv.at[j]], add=True)
    plsc.subcore_barrier()

    @pl.when(c == 0)
    def _():
        pltpu.sync_copy(osh.at[pl.ds(s * (OS // NT), OS // NT)],
                        oL.at[pl.ds(s * (OS // NT), OS // NT)])

    @pl.when(c == 1)
    def _():
        pltpu.sync_copy(osh.at[pl.ds(s * (OS // NT), OS // NT)],
                        oR.at[pl.ds(s * (OS // NT), OS // NT)])


@functools.cache
def _sc_read():
  return pl.kernel(
    _sc_read_body,
    mesh=_mesh(),
    compiler_params=pltpu.CompilerParams(needs_layout_passes=False, use_tc_tiling_on_sc=False),
    out_type=[
        jax.ShapeDtypeStruct((OS, H), f32),
        jax.ShapeDtypeStruct((OS, H), f32),
    ],
    scratch_types=[
        pltpu.VMEM((RPT, EB), i32),
        pltpu.VMEM((RPT, EB), i32),
        pltpu.VMEM((EB, H), f32),
        pltpu.VMEM((NP,), i32),
        pltpu.VMEM_SHARED((OS, H), f32),
        pltpu.SemaphoreType.DMA,
    ],
  )


def _prep_body(x_ref, w_ref, b_ref, m_ref, fL_ref, fR_ref, xL_ref, xR_ref):
    f = jnp.maximum(
        jnp.dot(x_ref[...], w_ref[...], preferred_element_type=f32) + b_ref[...],
        0.0)
    m = m_ref[...]
    fL_ref[...] = f[:, :H]
    fR_ref[...] = f[:, H:]
    xL_ref[...] = f[:, :H] * m
    xR_ref[...] = f[:, H:] * m


_tc_prep = pl.pallas_call(
    _prep_body,
    grid=(NP // BN,),
    in_specs=[
        pl.BlockSpec((BN, D), lambda i: (i, 0)),
        pl.BlockSpec((D, D), lambda i: (0, 0)),
        pl.BlockSpec((1, D), lambda i: (0, 0)),
        pl.BlockSpec((BN, 1), lambda i: (i, 0)),
    ],
    out_specs=[pl.BlockSpec((BN, H), lambda i: (i, 0))] * 4,
    out_shape=[jax.ShapeDtypeStruct((NP, H), f32)] * 4,
)


def _layer_body(cnt_ref, fL_ref, fR_ref, eL_ref, eR_ref, xL_ref, xR_ref,
                w1_ref, b1_ref, w2_ref, b2_ref, oL_ref, oR_ref):
    t = (cnt_ref[...] > 0.0).astype(f32)
    sL = eL_ref[...] + fL_ref[...] * t
    sR = eR_ref[...] + fR_ref[...] * t
    sfull = jnp.concatenate([sL, sR], axis=1)
    h = jnp.maximum(
        jnp.dot(sfull, w1_ref[...], preferred_element_type=f32) + b1_ref[...],
        0.0)
    h = jnp.maximum(
        jnp.dot(h, w2_ref[...], preferred_element_type=f32) + b2_ref[...],
        0.0)
    oL_ref[...] = xL_ref[...] + h[:, :H] * t
    oR_ref[...] = xR_ref[...] + h[:, H:] * t


_tc_layer = pl.pallas_call(
    _layer_body,
    grid=(NP // BN,),
    in_specs=[
        pl.BlockSpec((BN, 1), lambda i: (i, 0)),
        pl.BlockSpec((BN, H), lambda i: (i, 0)),
        pl.BlockSpec((BN, H), lambda i: (i, 0)),
        pl.BlockSpec((BN, H), lambda i: (i, 0)),
        pl.BlockSpec((BN, H), lambda i: (i, 0)),
        pl.BlockSpec((BN, H), lambda i: (i, 0)),
        pl.BlockSpec((BN, H), lambda i: (i, 0)),
        pl.BlockSpec((D, D), lambda i: (0, 0)),
        pl.BlockSpec((1, D), lambda i: (0, 0)),
        pl.BlockSpec((D, D), lambda i: (0, 0)),
        pl.BlockSpec((1, D), lambda i: (0, 0)),
    ],
    out_specs=[pl.BlockSpec((BN, H), lambda i: (i, 0))] * 2,
    out_shape=[jax.ShapeDtypeStruct((NP, H), f32)] * 2,
)


def _final_body(x_ref, w1_ref, b1_ref, w2_ref, b2_ref, o_ref):
    h = jnp.maximum(
        jnp.dot(x_ref[...], w1_ref[...], preferred_element_type=f32) + b1_ref[...],
        0.0)
    o_ref[...] = jnp.dot(h, w2_ref[...], preferred_element_type=f32) + b2_ref[...]


_tc_final = pl.pallas_call(
    _final_body,
    out_shape=jax.ShapeDtypeStruct((G, T), f32),
)


def kernel(dag_x, dag_edge_index, dag_layers_mask, dag_edge_attr, dag_readouts,
           dag_leaves, batch, Wt, bt, W1, b1, W2, b2, Wfc1, bfc1, Wfc2, bfc2):
    xpad = jnp.pad(dag_x, ((0, NP - N), (0, 0)))
    src2 = dag_edge_index[0].astype(i32).reshape(ER, EB)
    dst2 = dag_edge_index[1].astype(i32).reshape(ER, EB)
    msk2 = dag_layers_mask.astype(i32).reshape(ER, EB)
    att2 = dag_edge_attr.astype(f32).reshape(ER, EB)
    batchp = jnp.pad(batch.astype(i32), (0, NP - N))
    r2 = dag_readouts.astype(i32).reshape(RR, EB)
    leaves0 = dag_leaves[0].astype(i32)

    im, lcnt, psrc, pdst, patt, pcnt = _sc_prep()(src2, dst2, msk2, att2, leaves0)
    fL, fR, xL, xR = _tc_prep(xpad, Wt, bt.reshape(1, D), im.reshape(NP, 1))
    lcnt2 = lcnt.reshape(L, NP)

    for i in range(L):
        li = jnp.full((16,), i, i32)
        exL, exR = _sc_layer()(xL, xR, psrc, pdst, patt, pcnt, li)
        xL, xR = _tc_layer(lcnt2[i].reshape(NP, 1), fL, fR, exL, exR, xL, xR,
                           W1[i], b1[i].reshape(1, D), W2[i], b2[i].reshape(1, D))

    oL, oR = _sc_read()(xL, xR, r2, batchp)
    xcat = jnp.concatenate(
        [jnp.concatenate([oL[i * G:(i + 1) * G], oR[i * G:(i + 1) * G]], axis=1)
         for i in range(L + 1)], axis=1)
    out = _tc_final(xcat, Wfc1, bfc1.reshape(1, D), Wfc2, bfc2.reshape(1, T))
    return out
